# baseline (speedup 1.0000x reference)
import functools

import jax
import jax.numpy as jnp
from jax import lax
from jax.experimental import pallas as pl
from jax.experimental.pallas import tpu as pltpu
from jax.experimental.pallas import tpu_sc as plsc

_VOCAB = 100000
_DIM = 64
_BATCH = 16384
_NC = 2
_NS = 16
_L = 16
_NW = _NC * _NS
_BPW = _BATCH // _NW
_CHUNK = 256
_PHASES = _BPW // _CHUNK
_GROUPS = _CHUNK // _L


def _sc_body(s_hbm, d_hbm, head_hbm, tail_hbm, out_hbm,
             s_v, d_v, s2_v, d2_v, hrows_v, trows_v, out_v, sem_h, sem_t):
    wid = lax.axis_index("s") * _NC + lax.axis_index("c")
    base = wid * _BPW
    pltpu.sync_copy(s_hbm.at[pl.ds(base, _BPW)], s_v)
    pltpu.sync_copy(d_hbm.at[pl.ds(base, _BPW)], d_v)

    def halve_step(k, carry):
        s2_v[pl.ds(k * _L, _L)] = s_v[pl.ds(k * _L, _L)] >> 1
        d2_v[pl.ds(k * _L, _L)] = d_v[pl.ds(k * _L, _L)] >> 1
        return carry

    lax.fori_loop(0, _BPW // _L, halve_step, 0)

    def phase_step(p, carry):
        poff = p * _CHUNK
        ch = pltpu.async_copy(head_hbm.at[s2_v.at[pl.ds(poff, _CHUNK)]],
                              hrows_v, sem_h)
        ct = pltpu.async_copy(tail_hbm.at[d2_v.at[pl.ds(poff, _CHUNK)]],
                              trows_v, sem_t)
        ch.wait()
        ct.wait()

        def group_step(g, inner):
            rowlane = lax.iota(jnp.int32, _L) + g * _L
            svals = s_v[pl.ds(poff + g * _L, _L)]
            dvals = d_v[pl.ds(poff + g * _L, _L)]
            hoff = (svals & 1) * _DIM
            toff = (dvals & 1) * _DIM
            acc = jnp.zeros((_L,), jnp.float32)
            for j in range(_DIM):
                h = plsc.load_gather(hrows_v, [rowlane, hoff + j])
                t = plsc.load_gather(trows_v, [rowlane, toff + j])
                acc = acc + h * t
            out_v[pl.ds(poff + g * _L, _L)] = 1.0 / (1.0 + jnp.exp(acc))
            return inner

        lax.fori_loop(0, _GROUPS, group_step, 0)
        return carry

    lax.fori_loop(0, _PHASES, phase_step, 0)
    pltpu.sync_copy(out_v, out_hbm.at[pl.ds(base, _BPW)])


_sc_kernel = functools.partial(
    pl.kernel,
    out_type=jax.ShapeDtypeStruct((_BATCH,), jnp.float32),
    mesh=plsc.VectorSubcoreMesh(core_axis_name="c", subcore_axis_name="s",
                                num_cores=_NC, num_subcores=_NS),
    compiler_params=pltpu.CompilerParams(needs_layout_passes=False),
    scratch_types=[
        pltpu.VMEM((_BPW,), jnp.int32),
        pltpu.VMEM((_BPW,), jnp.int32),
        pltpu.VMEM((_BPW,), jnp.int32),
        pltpu.VMEM((_BPW,), jnp.int32),
        pltpu.VMEM((_CHUNK, 2 * _DIM), jnp.float32),
        pltpu.VMEM((_CHUNK, 2 * _DIM), jnp.float32),
        pltpu.VMEM((_BPW,), jnp.float32),
        pltpu.SemaphoreType.DMA,
        pltpu.SemaphoreType.DMA,
    ],
)(_sc_body)


def kernel(x, emb_head, emb_tail):
    s = x[:, 0]
    d = x[:, 1]
    h2 = emb_head.reshape(_VOCAB // 2, 2 * _DIM)
    t2 = emb_tail.reshape(_VOCAB // 2, 2 * _DIM)
    return _sc_kernel(s, d, h2, t2)


# speedup vs baseline: 1.0683x; 1.0683x over previous
import functools

import jax
import jax.numpy as jnp
from jax import lax
from jax.experimental import pallas as pl
from jax.experimental.pallas import tpu as pltpu
from jax.experimental.pallas import tpu_sc as plsc

_VOCAB = 100000
_DIM = 64
_ROW = 128
_BATCH = 16384
_NC = 2
_NS = 16
_L = 16
_NW = _NC * _NS
_BPW = _BATCH // _NW
_CHUNK = 256
_PHASES = _BPW // _CHUNK
_GROUPS = _CHUNK // _L


def _sc_body(s_hbm, d_hbm, head_hbm, tail_hbm, out_hbm,
             s_v, d_v, hrows_v, trows_v, out_v, sem_h, sem_t):
    wid = lax.axis_index("s") * _NC + lax.axis_index("c")
    base = wid * _BPW
    pltpu.sync_copy(s_hbm.at[pl.ds(base, _BPW)], s_v)
    pltpu.sync_copy(d_hbm.at[pl.ds(base, _BPW)], d_v)

    def phase_step(p, carry):
        poff = p * _CHUNK
        ch = pltpu.async_copy(head_hbm.at[s_v.at[pl.ds(poff, _CHUNK)]],
                              hrows_v, sem_h)
        ct = pltpu.async_copy(tail_hbm.at[d_v.at[pl.ds(poff, _CHUNK)]],
                              trows_v, sem_t)
        ch.wait()
        ct.wait()

        def group_step(g, inner):
            rowlane = lax.iota(jnp.int32, _L) + g * _L
            acc = jnp.zeros((_L,), jnp.float32)
            for j in range(_DIM):
                jv = jnp.full((_L,), j, jnp.int32)
                h = plsc.load_gather(hrows_v, [rowlane, jv])
                t = plsc.load_gather(trows_v, [rowlane, jv])
                acc = acc + h * t
            out_v[pl.ds(poff + g * _L, _L)] = 1.0 / (1.0 + jnp.exp(acc))
            return inner

        lax.fori_loop(0, _GROUPS, group_step, 0)
        return carry

    lax.fori_loop(0, _PHASES, phase_step, 0)
    pltpu.sync_copy(out_v, out_hbm.at[pl.ds(base, _BPW)])


_sc_kernel = functools.partial(
    pl.kernel,
    out_type=jax.ShapeDtypeStruct((_BATCH,), jnp.float32),
    mesh=plsc.VectorSubcoreMesh(core_axis_name="c", subcore_axis_name="s",
                                num_cores=_NC, num_subcores=_NS),
    compiler_params=pltpu.CompilerParams(needs_layout_passes=False),
    scratch_types=[
        pltpu.VMEM((_BPW,), jnp.int32),
        pltpu.VMEM((_BPW,), jnp.int32),
        pltpu.VMEM((_CHUNK, _ROW), jnp.float32),
        pltpu.VMEM((_CHUNK, _ROW), jnp.float32),
        pltpu.VMEM((_BPW,), jnp.float32),
        pltpu.SemaphoreType.DMA,
        pltpu.SemaphoreType.DMA,
    ],
)(_sc_body)


def kernel(x, emb_head, emb_tail):
    s = x[:, 0]
    d = x[:, 1]
    hp = jnp.pad(emb_head, ((0, 0), (0, _ROW - _DIM)))
    tp = jnp.pad(emb_tail, ((0, 0), (0, _ROW - _DIM)))
    return _sc_kernel(s, d, hp, tp)


# speedup vs baseline: 1.2310x; 1.1522x over previous
import functools

import jax
import jax.numpy as jnp
from jax import lax
from jax.experimental import pallas as pl
from jax.experimental.pallas import tpu as pltpu
from jax.experimental.pallas import tpu_sc as plsc

_VOCAB = 100000
_DIM = 64
_BATCH = 16384
_NC = 2
_NS = 16
_L = 16
_NW = _NC * _NS
_BPW = _BATCH // _NW
_GROUPS = _BPW // _L


def _sc_body(s_hbm, d_hbm, head_hbm, tail_hbm, out_hbm,
             s_v, d_v, hrows_v, trows_v, out_v, sem_h, sem_t):
    wid = lax.axis_index("s") * _NC + lax.axis_index("c")
    base = wid * _BPW
    pltpu.sync_copy(s_hbm.at[pl.ds(base, _BPW)], s_v)
    pltpu.sync_copy(d_hbm.at[pl.ds(base, _BPW)], d_v)
    ch = pltpu.async_copy(head_hbm.at[s_v], hrows_v, sem_h)
    ct = pltpu.async_copy(tail_hbm.at[d_v], trows_v, sem_t)
    ch.wait()
    ct.wait()

    lid = lax.iota(jnp.int32, _L)

    def group_step(g, carry):
        pbase = g * _L
        res = jnp.zeros((_L,), jnp.float32)
        for p in range(_L):
            row = pbase + p
            prod = (hrows_v[row, pl.ds(0, _L)] * trows_v[row, pl.ds(0, _L)]
                    + hrows_v[row, pl.ds(_L, _L)] * trows_v[row, pl.ds(_L, _L)]
                    + hrows_v[row, pl.ds(2 * _L, _L)] * trows_v[row, pl.ds(2 * _L, _L)]
                    + hrows_v[row, pl.ds(3 * _L, _L)] * trows_v[row, pl.ds(3 * _L, _L)])
            res = jnp.where(lid == p, jnp.sum(prod), res)
        out_v[pl.ds(pbase, _L)] = 1.0 / (1.0 + jnp.exp(res))
        return carry

    lax.fori_loop(0, _GROUPS, group_step, 0)
    pltpu.sync_copy(out_v, out_hbm.at[pl.ds(base, _BPW)])


_sc_kernel = functools.partial(
    pl.kernel,
    out_type=jax.ShapeDtypeStruct((_BATCH,), jnp.float32),
    mesh=plsc.VectorSubcoreMesh(core_axis_name="c", subcore_axis_name="s",
                                num_cores=_NC, num_subcores=_NS),
    compiler_params=pltpu.CompilerParams(needs_layout_passes=False,
                                         use_tc_tiling_on_sc=False),
    scratch_types=[
        pltpu.VMEM((_BPW,), jnp.int32),
        pltpu.VMEM((_BPW,), jnp.int32),
        pltpu.VMEM((_BPW, _DIM), jnp.float32),
        pltpu.VMEM((_BPW, _DIM), jnp.float32),
        pltpu.VMEM((_BPW,), jnp.float32),
        pltpu.SemaphoreType.DMA,
        pltpu.SemaphoreType.DMA,
    ],
)(_sc_body)


def kernel(x, emb_head, emb_tail):
    s = x[:, 0]
    d = x[:, 1]
    return _sc_kernel(s, d, emb_head, emb_tail)
